# Initial kernel scaffold; baseline (speedup 1.0000x reference)
import jax
import jax.numpy as jnp
from jax.experimental import pallas as pl


def kernel(x, adj_t, W_l1, b_l1, W_r1, W_l2, b_l2, W_r2, W_l3, b_l3, W_r3):
    raise NotImplementedError("write your pallas kernel here")


# speedup vs baseline: 5.6273x; 5.6273x over previous
import functools

import jax
import jax.numpy as jnp
from jax import lax
from jax.experimental import pallas as pl
from jax.experimental.pallas import tpu as pltpu
from jax.experimental.pallas import tpu_sc as plsc

N = 10000
E = 320000
D = 128

NC = 2
NS = 16
NW = NC * NS
EPW = E // NW
CHUNK = 80
NCHUNK = EPW // CHUNK
GCH = 5
NGROUP = NCHUNK // GCH
STRIPE = 624
TAIL = N - NS * STRIPE
ZBLK = 24

_MESH = plsc.VectorSubcoreMesh(core_axis_name="c", subcore_axis_name="s")


def _fill_block(ref, rows, value):
    def fill(i, _):
        v = jnp.full((16,), value, jnp.float32)
        for c in range(D // 16):
            ref[i, pl.ds(c * 16, 16)] = v
        return 0
    lax.fori_loop(0, rows, fill, 0)


def _zero_stripe(acc_sh, zero_v, sid):
    base = pl.multiple_of(sid * STRIPE, 8)
    for k in range(STRIPE // ZBLK):
        off = pl.multiple_of(base + k * ZBLK, 8)
        pltpu.sync_copy(zero_v, acc_sh.at[pl.ds(off, ZBLK)])

    @pl.when(sid == NS - 1)
    def _zero_tail():
        pltpu.sync_copy(zero_v.at[pl.ds(0, TAIL)],
                        acc_sh.at[pl.ds(NS * STRIPE, TAIL)])
    return base


def _write_stripe(acc_sh, out_hbm, cid, sid, base):
    pltpu.sync_copy(acc_sh.at[pl.ds(base, STRIPE)],
                    out_hbm.at[cid, pl.ds(base, STRIPE)])

    @pl.when(sid == NS - 1)
    def _out_tail():
        pltpu.sync_copy(acc_sh.at[pl.ds(NS * STRIPE, TAIL)],
                        out_hbm.at[cid, pl.ds(NS * STRIPE, TAIL)])


@functools.partial(
    pl.kernel,
    out_type=jax.ShapeDtypeStruct((NC, N, D), jnp.float32),
    mesh=_MESH,
    scratch_types=[
        pltpu.VMEM_SHARED((N, D), jnp.float32),
        pltpu.VMEM((GCH, CHUNK), jnp.int32),
        pltpu.VMEM((GCH, CHUNK), jnp.int32),
        pltpu.VMEM((CHUNK, D), jnp.float32),
        pltpu.VMEM((ZBLK, D), jnp.float32),
        pltpu.SemaphoreType.DMA,
    ],
)
def _sc_agg(x_hbm, src_hbm, dst_hbm, agg_hbm,
            acc_sh, src_v, dst_v, rows_v, zero_v, sem):
    cid = lax.axis_index("c")
    sid = lax.axis_index("s")
    wid = cid * NS + sid

    _fill_block(zero_v, ZBLK, 0.0)
    base = _zero_stripe(acc_sh, zero_v, sid)
    plsc.subcore_barrier()

    def group(g, _):
        pltpu.sync_copy(src_hbm.at[wid, g], src_v)
        pltpu.sync_copy(dst_hbm.at[wid, g], dst_v)

        def step(j, _):
            pltpu.async_copy(x_hbm.at[src_v.at[j]], rows_v, sem).wait()
            pltpu.sync_copy(rows_v, acc_sh.at[dst_v.at[j]], add=True)
            return 0
        lax.fori_loop(0, GCH, step, 0)
        return 0
    lax.fori_loop(0, NGROUP, group, 0)
    plsc.subcore_barrier()

    _write_stripe(acc_sh, agg_hbm, cid, sid, base)


@functools.partial(
    pl.kernel,
    out_type=jax.ShapeDtypeStruct((NC, N, D), jnp.float32),
    mesh=_MESH,
    scratch_types=[
        pltpu.VMEM_SHARED((N, D), jnp.float32),
        pltpu.VMEM((GCH, CHUNK), jnp.int32),
        pltpu.VMEM((CHUNK, D), jnp.float32),
        pltpu.VMEM((ZBLK, D), jnp.float32),
    ],
)
def _sc_cnt(dst_hbm, cnt_hbm, acc_sh, dst_v, ones_v, zero_v):
    cid = lax.axis_index("c")
    sid = lax.axis_index("s")
    wid = cid * NS + sid

    _fill_block(zero_v, ZBLK, 0.0)
    _fill_block(ones_v, CHUNK, 1.0)
    base = _zero_stripe(acc_sh, zero_v, sid)
    plsc.subcore_barrier()

    def group(g, _):
        pltpu.sync_copy(dst_hbm.at[wid, g], dst_v)

        def step(j, _):
            pltpu.sync_copy(ones_v, acc_sh.at[dst_v.at[j]], add=True)
            return 0
        lax.fori_loop(0, GCH, step, 0)
        return 0
    lax.fori_loop(0, NGROUP, group, 0)
    plsc.subcore_barrier()

    _write_stripe(acc_sh, cnt_hbm, cid, sid, base)


BLK = 400


def _make_tc_dense(relu: bool):
    def body(agg_ref, cnt_ref, x_ref, wl_ref, wr_ref, b_ref, o_ref):
        agg = agg_ref[0] + agg_ref[1]
        cnt = cnt_ref[0][:, 0:1] + cnt_ref[1][:, 0:1]
        mean = agg * (1.0 / jnp.maximum(cnt, 1.0))
        out = (jnp.dot(mean, wl_ref[...], preferred_element_type=jnp.float32)
               + jnp.dot(x_ref[...], wr_ref[...], preferred_element_type=jnp.float32)
               + b_ref[...])
        if relu:
            out = jnp.maximum(out, 0.0)
        o_ref[...] = out

    return pl.pallas_call(
        body,
        grid=(N // BLK,),
        in_specs=[
            pl.BlockSpec((NC, BLK, D), lambda i: (0, i, 0)),
            pl.BlockSpec((NC, BLK, D), lambda i: (0, i, 0)),
            pl.BlockSpec((BLK, D), lambda i: (i, 0)),
            pl.BlockSpec((D, D), lambda i: (0, 0)),
            pl.BlockSpec((D, D), lambda i: (0, 0)),
            pl.BlockSpec((1, D), lambda i: (0, 0)),
        ],
        out_specs=pl.BlockSpec((BLK, D), lambda i: (i, 0)),
        out_shape=jax.ShapeDtypeStruct((N, D), jnp.float32),
    )


_tc_dense_relu = _make_tc_dense(True)
_tc_dense = _make_tc_dense(False)


def kernel(x, adj_t, W_l1, b_l1, W_r1, W_l2, b_l2, W_r2, W_l3, b_l3, W_r3):
    src = adj_t[0].reshape(NW, NGROUP, GCH, CHUNK)
    dst = adj_t[1].reshape(NW, NGROUP, GCH, CHUNK)

    cnt = _sc_cnt(dst)
    agg1 = _sc_agg(x, src, dst)
    h1 = _tc_dense_relu(agg1, cnt, x, W_l1, W_r1, b_l1.reshape(1, D))
    agg2 = _sc_agg(h1, src, dst)
    h2 = _tc_dense_relu(agg2, cnt, h1, W_l2, W_r2, b_l2.reshape(1, D))
    agg3 = _sc_agg(h2, src, dst)
    h3 = _tc_dense(agg3, cnt, h2, W_l3, W_r3, b_l3.reshape(1, D))
    return h3


# speedup vs baseline: 7.7176x; 1.3715x over previous
import functools

import jax
import jax.numpy as jnp
from jax import lax
from jax.experimental import pallas as pl
from jax.experimental.pallas import tpu as pltpu
from jax.experimental.pallas import tpu_sc as plsc

N = 10000
E = 320000
D = 128

NC = 2
NS = 16
NW = NC * NS
EPW = E // NW
CHUNK = 100
NCHUNK = EPW // CHUNK
GCH = 10
NGROUP = NCHUNK // GCH
STRIPE = 624
TAIL = N - NS * STRIPE
ZBLK = 24

_MESH = plsc.VectorSubcoreMesh(core_axis_name="c", subcore_axis_name="s")


def _fill_block(ref, rows, value):
    def fill(i, _):
        v = jnp.full((16,), value, jnp.float32)
        for c in range(D // 16):
            ref[i, pl.ds(c * 16, 16)] = v
        return 0
    lax.fori_loop(0, rows, fill, 0)


def _zero_stripe(acc_sh, zero_v, sid):
    base = pl.multiple_of(sid * STRIPE, 8)
    for k in range(STRIPE // ZBLK):
        off = pl.multiple_of(base + k * ZBLK, 8)
        pltpu.sync_copy(zero_v, acc_sh.at[pl.ds(off, ZBLK)])

    @pl.when(sid == NS - 1)
    def _zero_tail():
        pltpu.sync_copy(zero_v.at[pl.ds(0, TAIL)],
                        acc_sh.at[pl.ds(NS * STRIPE, TAIL)])
    return base


def _write_stripe(acc_sh, out_hbm, cid, sid, base):
    pltpu.sync_copy(acc_sh.at[pl.ds(base, STRIPE)],
                    out_hbm.at[cid, pl.ds(base, STRIPE)])

    @pl.when(sid == NS - 1)
    def _out_tail():
        pltpu.sync_copy(acc_sh.at[pl.ds(NS * STRIPE, TAIL)],
                        out_hbm.at[cid, pl.ds(NS * STRIPE, TAIL)])


@functools.partial(
    pl.kernel,
    out_type=jax.ShapeDtypeStruct((NC, N, D), jnp.float32),
    mesh=_MESH,
    scratch_types=[
        pltpu.VMEM_SHARED((N, D), jnp.float32),
        pltpu.VMEM((GCH, CHUNK), jnp.int32),
        pltpu.VMEM((GCH, CHUNK), jnp.int32),
        pltpu.VMEM((CHUNK, D), jnp.float32),
        pltpu.VMEM((CHUNK, D), jnp.float32),
        pltpu.VMEM((ZBLK, D), jnp.float32),
        pltpu.SemaphoreType.DMA,
        pltpu.SemaphoreType.DMA,
    ],
)
def _sc_agg(x_hbm, src_hbm, dst_hbm, agg_hbm,
            acc_sh, src_v, dst_v, rows0_v, rows1_v, zero_v, sem0, sem1):
    cid = lax.axis_index("c")
    sid = lax.axis_index("s")
    wid = cid * NS + sid

    _fill_block(zero_v, ZBLK, 0.0)
    base = _zero_stripe(acc_sh, zero_v, sid)
    plsc.subcore_barrier()

    bufs = (rows0_v, rows1_v)
    sems = (sem0, sem1)

    def group(g, _):
        pltpu.sync_copy(src_hbm.at[wid, g], src_v)
        pltpu.sync_copy(dst_hbm.at[wid, g], dst_v)
        pending = [None, None]
        pending[0] = pltpu.async_copy(x_hbm.at[src_v.at[0]], bufs[0], sems[0])
        for j in range(GCH):
            b = j % 2
            pending[b].wait()
            if j + 1 < GCH:
                nb = (j + 1) % 2
                pending[nb] = pltpu.async_copy(
                    x_hbm.at[src_v.at[j + 1]], bufs[nb], sems[nb])
            pltpu.sync_copy(bufs[b], acc_sh.at[dst_v.at[j]], add=True)
        return 0
    lax.fori_loop(0, NGROUP, group, 0)
    plsc.subcore_barrier()

    _write_stripe(acc_sh, agg_hbm, cid, sid, base)


@functools.partial(
    pl.kernel,
    out_type=jax.ShapeDtypeStruct((NC, N, D), jnp.float32),
    mesh=_MESH,
    scratch_types=[
        pltpu.VMEM_SHARED((N, D), jnp.float32),
        pltpu.VMEM((GCH, CHUNK), jnp.int32),
        pltpu.VMEM((CHUNK, D), jnp.float32),
        pltpu.VMEM((ZBLK, D), jnp.float32),
    ],
)
def _sc_cnt(dst_hbm, cnt_hbm, acc_sh, dst_v, ones_v, zero_v):
    cid = lax.axis_index("c")
    sid = lax.axis_index("s")
    wid = cid * NS + sid

    _fill_block(zero_v, ZBLK, 0.0)
    _fill_block(ones_v, CHUNK, 1.0)
    base = _zero_stripe(acc_sh, zero_v, sid)
    plsc.subcore_barrier()

    def group(g, _):
        pltpu.sync_copy(dst_hbm.at[wid, g], dst_v)

        def step(j, _):
            pltpu.sync_copy(ones_v, acc_sh.at[dst_v.at[j]], add=True)
            return 0
        lax.fori_loop(0, GCH, step, 0)
        return 0
    lax.fori_loop(0, NGROUP, group, 0)
    plsc.subcore_barrier()

    _write_stripe(acc_sh, cnt_hbm, cid, sid, base)


BLK = 400


def _make_tc_dense(relu: bool):
    def body(agg_ref, cnt_ref, x_ref, wl_ref, wr_ref, b_ref, o_ref):
        agg = agg_ref[0] + agg_ref[1]
        cnt = cnt_ref[0][:, 0:1] + cnt_ref[1][:, 0:1]
        mean = agg * (1.0 / jnp.maximum(cnt, 1.0))
        out = (jnp.dot(mean, wl_ref[...], preferred_element_type=jnp.float32)
               + jnp.dot(x_ref[...], wr_ref[...], preferred_element_type=jnp.float32)
               + b_ref[...])
        if relu:
            out = jnp.maximum(out, 0.0)
        o_ref[...] = out

    return pl.pallas_call(
        body,
        grid=(N // BLK,),
        in_specs=[
            pl.BlockSpec((NC, BLK, D), lambda i: (0, i, 0)),
            pl.BlockSpec((NC, BLK, D), lambda i: (0, i, 0)),
            pl.BlockSpec((BLK, D), lambda i: (i, 0)),
            pl.BlockSpec((D, D), lambda i: (0, 0)),
            pl.BlockSpec((D, D), lambda i: (0, 0)),
            pl.BlockSpec((1, D), lambda i: (0, 0)),
        ],
        out_specs=pl.BlockSpec((BLK, D), lambda i: (i, 0)),
        out_shape=jax.ShapeDtypeStruct((N, D), jnp.float32),
    )


_tc_dense_relu = _make_tc_dense(True)
_tc_dense = _make_tc_dense(False)


def kernel(x, adj_t, W_l1, b_l1, W_r1, W_l2, b_l2, W_r2, W_l3, b_l3, W_r3):
    src = adj_t[0].reshape(NW, NGROUP, GCH, CHUNK)
    dst = adj_t[1].reshape(NW, NGROUP, GCH, CHUNK)

    cnt = _sc_cnt(dst)
    agg1 = _sc_agg(x, src, dst)
    h1 = _tc_dense_relu(agg1, cnt, x, W_l1, W_r1, b_l1.reshape(1, D))
    agg2 = _sc_agg(h1, src, dst)
    h2 = _tc_dense_relu(agg2, cnt, h1, W_l2, W_r2, b_l2.reshape(1, D))
    agg3 = _sc_agg(h2, src, dst)
    h3 = _tc_dense(agg3, cnt, h2, W_l3, W_r3, b_l3.reshape(1, D))
    return h3
